# Initial kernel scaffold; baseline (speedup 1.0000x reference)
import jax
import jax.numpy as jnp
from jax.experimental import pallas as pl


def kernel(x, edge_index, batch, W1, b1, W2, b2, Wo1, bo1, Wo2, bo2):
    raise NotImplementedError("write your pallas kernel here")


# speedup vs baseline: 21.3252x; 21.3252x over previous
import functools

import jax
import jax.numpy as jnp
from jax import lax
from jax.experimental import pallas as pl
from jax.experimental.pallas import tpu as pltpu
from jax.experimental.pallas import tpu_sc as plsc

N = 100000
E = 1600000
IN_DIM = 11
HID = 32
HALF = 16
OUT_DIM = 1

NC = 2
NS = 16
LANES = 128
ROWS = 12800
EPAD = ROWS * LANES - E

G = 8
RPT = ROWS // NS
NGP = RPT // G
RPT_D = ROWS // (NC * NS)
NGD = RPT_D // G

NPAD = 16
NPT = 6256
NCH = 368
NKC = NPT // NCH

_mesh = plsc.VectorSubcoreMesh(
    core_axis_name="c", subcore_axis_name="s", num_cores=NC, num_subcores=NS
)


@functools.partial(
    pl.kernel,
    out_type=[jax.ShapeDtypeStruct((N,), jnp.float32),
              jax.ShapeDtypeStruct((N,), jnp.float32)],
    mesh=_mesh,
    scratch_types=[
        pltpu.VMEM((G, LANES), jnp.int32),
        pltpu.VMEM((LANES,), jnp.float32),
        pltpu.VMEM((NCH,), jnp.float32),
        pltpu.VMEM_SHARED((N + NPAD,), jnp.float32),
        pltpu.SemaphoreType.DMA,
    ],
    compiler_params=pltpu.CompilerParams(use_tc_tiling_on_sc=False),
)
def _deg_kernel(dst_hbm, outa_hbm, outb_hbm, dstb, ones, obuf, dacc, ssem):
    c = lax.axis_index("c")
    s = lax.axis_index("s")
    nbase = jnp.minimum(s * NPT, N - NPT)

    def _zero(i, carry):
        obuf[pl.ds(i * 16, 16)] = jnp.zeros((16,), jnp.float32)
        return carry

    lax.fori_loop(0, NCH // 16, _zero, 0)

    def _one(i, carry):
        ones[pl.ds(i * 16, 16)] = jnp.ones((16,), jnp.float32)
        return carry

    lax.fori_loop(0, LANES // 16, _one, 0)

    for k in range(NKC):
        pltpu.sync_copy(obuf, dacc.at[pl.ds(nbase + k * NCH, NCH)])
    plsc.subcore_barrier()

    rbase = (c * NS + s) * RPT_D

    def _group(g, carry):
        pltpu.sync_copy(dst_hbm.at[pl.ds(rbase + g * G, G)], dstb)

        def _fire(j, cc):
            pltpu.async_copy(ones, dacc.at[dstb.at[j]], ssem, add=True)
            return cc

        lax.fori_loop(0, G, _fire, 0)

        def _drain(j, cc):
            pltpu.make_async_copy(ones, dacc.at[dstb.at[j]], ssem).wait()
            return cc

        lax.fori_loop(0, G, _drain, 0)
        return carry

    lax.fori_loop(0, NGD, _group, 0)
    plsc.subcore_barrier()

    for k in range(NKC):
        sl = pl.ds(nbase + k * NCH, NCH)
        pltpu.sync_copy(dacc.at[sl], obuf)

        @pl.when(c == 0)
        def _():
            pltpu.sync_copy(obuf, outa_hbm.at[sl])

        @pl.when(c == 1)
        def _():
            pltpu.sync_copy(obuf, outb_hbm.at[sl])


@functools.partial(
    pl.kernel,
    out_type=[
        jax.ShapeDtypeStruct((N, HALF), jnp.float32),
        jax.ShapeDtypeStruct((N, HALF), jnp.float32),
    ],
    mesh=_mesh,
    scratch_types=[
        pltpu.VMEM((G, LANES), jnp.int32),
        pltpu.VMEM((G, LANES), jnp.int32),
        pltpu.VMEM((G, LANES, HALF), jnp.float32),
        pltpu.VMEM((NCH, HALF), jnp.float32),
        pltpu.VMEM_SHARED((N + NPAD, HALF), jnp.float32),
        pltpu.SemaphoreType.DMA,
        pltpu.SemaphoreType.DMA,
    ],
    compiler_params=pltpu.CompilerParams(use_tc_tiling_on_sc=False),
)
def _prop_kernel(src_hbm, dst_hbm, ga_hbm, gb_hbm, pa_hbm, pb_hbm,
                 srcb, dstb, rows, obuf, acc, gsem, ssem):
    c = lax.axis_index("c")
    s = lax.axis_index("s")
    nbase = jnp.minimum(s * NPT, N - NPT)

    def _zero(i, carry):
        obuf[i] = jnp.zeros((16,), jnp.float32)
        return carry

    lax.fori_loop(0, NCH, _zero, 0)

    for k in range(NKC):
        pltpu.sync_copy(obuf, acc.at[pl.ds(nbase + k * NCH, NCH)])
    plsc.subcore_barrier()

    def _run(g_hbm, p_hbm):
        def _group(g, carry):
            base = s * RPT + g * G
            pltpu.sync_copy(src_hbm.at[pl.ds(base, G)], srcb)
            pltpu.sync_copy(dst_hbm.at[pl.ds(base, G)], dstb)

            def _fire_g(j, cc):
                pltpu.async_copy(g_hbm.at[srcb.at[j]], rows.at[j], gsem)
                return cc

            lax.fori_loop(0, G, _fire_g, 0)

            def _drain_g(j, cc):
                pltpu.make_async_copy(g_hbm.at[srcb.at[j]], rows.at[j], gsem).wait()
                return cc

            lax.fori_loop(0, G, _drain_g, 0)

            def _fire_s(j, cc):
                pltpu.async_copy(rows.at[j], acc.at[dstb.at[j]], ssem, add=True)
                return cc

            lax.fori_loop(0, G, _fire_s, 0)

            def _drain_s(j, cc):
                pltpu.make_async_copy(rows.at[j], acc.at[dstb.at[j]], ssem).wait()
                return cc

            lax.fori_loop(0, G, _drain_s, 0)
            return carry

        lax.fori_loop(0, NGP, _group, 0)
        plsc.subcore_barrier()

        for k in range(NKC):
            sl = pl.ds(nbase + k * NCH, NCH)
            pltpu.sync_copy(acc.at[sl], obuf)
            pltpu.sync_copy(obuf, p_hbm.at[sl])

    @pl.when(c == 0)
    def _():
        _run(ga_hbm, pa_hbm)

    @pl.when(c == 1)
    def _():
        _run(gb_hbm, pb_hbm)


R = 2000


def _k1_body(x_ref, dga_ref, dgb_ref, w1_ref, ga_ref, gb_ref):
    d = lax.rsqrt(dga_ref[:, 0] + dgb_ref[:, 0] + 1.0)
    h = jnp.dot(x_ref[...], w1_ref[...], preferred_element_type=jnp.float32)
    g = h * d[:, None]
    ga_ref[...] = g[:, :HALF]
    gb_ref[...] = g[:, HALF:]


def _mid_body(pa_ref, pb_ref, ga_ref, gb_ref, dga_ref, dgb_ref, b_ref, w_ref,
              oa_ref, ob_ref):
    d = lax.rsqrt(dga_ref[:, 0] + dgb_ref[:, 0] + 1.0)
    za = jnp.maximum((pa_ref[...] + ga_ref[...]) * d[:, None] + b_ref[0, :HALF], 0.0)
    zb = jnp.maximum((pb_ref[...] + gb_ref[...]) * d[:, None] + b_ref[0, HALF:], 0.0)
    z = jnp.concatenate([za, zb], axis=1)
    h = jnp.dot(z, w_ref[...], preferred_element_type=jnp.float32)
    g = h * d[:, None]
    oa_ref[...] = g[:, :HALF]
    ob_ref[...] = g[:, HALF:]


def _head_body(pa_ref, pb_ref, ga_ref, gb_ref, dga_ref, dgb_ref, b2_ref,
               wo1_ref, bo1_ref, wo2_ref, bo2_ref, y_ref):
    d = lax.rsqrt(dga_ref[:, 0] + dgb_ref[:, 0] + 1.0)
    za = jnp.maximum((pa_ref[...] + ga_ref[...]) * d[:, None] + b2_ref[0, :HALF], 0.0)
    zb = jnp.maximum((pb_ref[...] + gb_ref[...]) * d[:, None] + b2_ref[0, HALF:], 0.0)
    z = jnp.concatenate([za, zb], axis=1)
    m = jnp.dot(z, wo1_ref[...], preferred_element_type=jnp.float32) + bo1_ref[0, :]
    m = jnp.where(m > 0.0, m, jnp.exp(jnp.minimum(m, 0.0)) - 1.0)
    y_ref[...] = jnp.dot(m, wo2_ref[...], preferred_element_type=jnp.float32) + bo2_ref[0, :]


def _row_spec(cols):
    return pl.BlockSpec((R, cols), lambda i: (i, 0))


def _full_spec(shape):
    nd = len(shape)
    return pl.BlockSpec(shape, lambda i: (0,) * nd)


def kernel(x, edge_index, batch, W1, b1, W2, b2, Wo1, bo1, Wo2, bo2):
    src = edge_index[0]
    dst = edge_index[1]
    src2d = jnp.concatenate([src, jnp.zeros((EPAD,), jnp.int32)]).reshape(ROWS, LANES)
    dst2d = jnp.concatenate([dst, jnp.full((EPAD,), N, jnp.int32)]).reshape(ROWS, LANES)

    dga, dgb = _deg_kernel(dst2d)
    dga = dga.reshape(N, 1)
    dgb = dgb.reshape(N, 1)

    ga, gb = pl.pallas_call(
        _k1_body,
        grid=(N // R,),
        in_specs=[_row_spec(IN_DIM), _row_spec(1), _row_spec(1),
                  _full_spec((IN_DIM, HID))],
        out_specs=[_row_spec(HALF), _row_spec(HALF)],
        out_shape=[jax.ShapeDtypeStruct((N, HALF), jnp.float32),
                   jax.ShapeDtypeStruct((N, HALF), jnp.float32)],
    )(x, dga, dgb, W1)

    pa, pb = _prop_kernel(src2d, dst2d, ga, gb)

    g2a, g2b = pl.pallas_call(
        _mid_body,
        grid=(N // R,),
        in_specs=[_row_spec(HALF), _row_spec(HALF), _row_spec(HALF),
                  _row_spec(HALF), _row_spec(1), _row_spec(1),
                  _full_spec((1, HID)), _full_spec((HID, HID))],
        out_specs=[_row_spec(HALF), _row_spec(HALF)],
        out_shape=[jax.ShapeDtypeStruct((N, HALF), jnp.float32),
                   jax.ShapeDtypeStruct((N, HALF), jnp.float32)],
    )(pa, pb, ga, gb, dga, dgb, b1.reshape(1, HID), W2)

    pa2, pb2 = _prop_kernel(src2d, dst2d, g2a, g2b)

    y = pl.pallas_call(
        _head_body,
        grid=(N // R,),
        in_specs=[_row_spec(HALF), _row_spec(HALF), _row_spec(HALF),
                  _row_spec(HALF), _row_spec(1), _row_spec(1),
                  _full_spec((1, HID)), _full_spec((HID, HALF)),
                  _full_spec((1, HALF)), _full_spec((HALF, OUT_DIM)),
                  _full_spec((1, OUT_DIM))],
        out_specs=[_row_spec(OUT_DIM)],
        out_shape=[jax.ShapeDtypeStruct((N, OUT_DIM), jnp.float32)],
    )(pa2, pb2, g2a, g2b, dga, dgb, b2.reshape(1, HID),
      Wo1, bo1.reshape(1, HALF), Wo2, bo2.reshape(1, OUT_DIM))[0]

    return jnp.squeeze(y, -1)


# speedup vs baseline: 24.1284x; 1.1314x over previous
import functools

import jax
import jax.numpy as jnp
from jax import lax
from jax.experimental import pallas as pl
from jax.experimental.pallas import tpu as pltpu
from jax.experimental.pallas import tpu_sc as plsc

N = 100000
E = 1600000
IN_DIM = 11
HID = 32
HALF = 16
OUT_DIM = 1

NC = 2
NS = 16
LANES = 128
ROWS = 12800
EPAD = ROWS * LANES - E

G = 8
RPT = ROWS // NS
RPT_D = ROWS // (NC * NS)
NGD = RPT_D // G

GB = 8
GH = 4
NB = RPT // GB

NPAD = 16
NPT = 6256
NCH = 368
NKC = NPT // NCH

_mesh = plsc.VectorSubcoreMesh(
    core_axis_name="c", subcore_axis_name="s", num_cores=NC, num_subcores=NS
)


@functools.partial(
    pl.kernel,
    out_type=[jax.ShapeDtypeStruct((N,), jnp.float32),
              jax.ShapeDtypeStruct((N,), jnp.float32)],
    mesh=_mesh,
    scratch_types=[
        pltpu.VMEM((G, LANES), jnp.int32),
        pltpu.VMEM((LANES,), jnp.float32),
        pltpu.VMEM((NCH,), jnp.float32),
        pltpu.VMEM_SHARED((N + NPAD,), jnp.float32),
        pltpu.SemaphoreType.DMA,
    ],
    compiler_params=pltpu.CompilerParams(use_tc_tiling_on_sc=False),
)
def _deg_kernel(dst_hbm, outa_hbm, outb_hbm, dstb, ones, obuf, dacc, ssem):
    c = lax.axis_index("c")
    s = lax.axis_index("s")
    nbase = jnp.minimum(s * NPT, N - NPT)

    def _zero(i, carry):
        obuf[pl.ds(i * 16, 16)] = jnp.zeros((16,), jnp.float32)
        return carry

    lax.fori_loop(0, NCH // 16, _zero, 0)

    def _one(i, carry):
        ones[pl.ds(i * 16, 16)] = jnp.ones((16,), jnp.float32)
        return carry

    lax.fori_loop(0, LANES // 16, _one, 0)

    for k in range(NKC):
        pltpu.sync_copy(obuf, dacc.at[pl.ds(nbase + k * NCH, NCH)])
    plsc.subcore_barrier()

    rbase = (c * NS + s) * RPT_D

    def _group(g, carry):
        pltpu.sync_copy(dst_hbm.at[pl.ds(rbase + g * G, G)], dstb)

        def _fire(j, cc):
            pltpu.async_copy(ones, dacc.at[dstb.at[j]], ssem, add=True)
            return cc

        lax.fori_loop(0, G, _fire, 0)

        def _drain(j, cc):
            pltpu.make_async_copy(ones, dacc.at[dstb.at[j]], ssem).wait()
            return cc

        lax.fori_loop(0, G, _drain, 0)
        return carry

    lax.fori_loop(0, NGD, _group, 0)
    plsc.subcore_barrier()

    for k in range(NKC):
        sl = pl.ds(nbase + k * NCH, NCH)
        pltpu.sync_copy(dacc.at[sl], obuf)

        @pl.when(c == 0)
        def _():
            pltpu.sync_copy(obuf, outa_hbm.at[sl])

        @pl.when(c == 1)
        def _():
            pltpu.sync_copy(obuf, outb_hbm.at[sl])


@functools.partial(
    pl.kernel,
    out_type=[
        jax.ShapeDtypeStruct((N, HALF), jnp.float32),
        jax.ShapeDtypeStruct((N, HALF), jnp.float32),
    ],
    mesh=_mesh,
    scratch_types=[
        pltpu.VMEM((2, GB, LANES), jnp.int32),
        pltpu.VMEM((2, GB, LANES), jnp.int32),
        pltpu.VMEM((GH, LANES, HALF), jnp.float32),
        pltpu.VMEM((GH, LANES, HALF), jnp.float32),
        pltpu.VMEM((NCH, HALF), jnp.float32),
        pltpu.VMEM_SHARED((N + NPAD, HALF), jnp.float32),
        pltpu.SemaphoreType.DMA,
        pltpu.SemaphoreType.DMA,
    ],
    compiler_params=pltpu.CompilerParams(use_tc_tiling_on_sc=False),
)
def _prop_kernel(src_hbm, dst_hbm, ga_hbm, gb_hbm, pa_hbm, pb_hbm,
                 ibs, ibd, rows_a, rows_b, obuf, acc, gsem, ssem):
    c = lax.axis_index("c")
    s = lax.axis_index("s")
    nbase = jnp.minimum(s * NPT, N - NPT)

    def _zero(i, carry):
        obuf[i] = jnp.zeros((16,), jnp.float32)
        return carry

    lax.fori_loop(0, NCH, _zero, 0)

    for k in range(NKC):
        pltpu.sync_copy(obuf, acc.at[pl.ds(nbase + k * NCH, NCH)])
    plsc.subcore_barrier()

    def _run(g_hbm, p_hbm):
        rbase = s * RPT

        def _load_idx(b, p):
            pltpu.sync_copy(src_hbm.at[pl.ds(rbase + b * GB, GB)], ibs.at[p])
            pltpu.sync_copy(dst_hbm.at[pl.ds(rbase + b * GB, GB)], ibd.at[p])

        def _fire_g(p, half, rbuf):
            def _f(j, cc):
                pltpu.async_copy(g_hbm.at[ibs.at[p, half * GH + j]],
                                 rbuf.at[j], gsem)
                return cc
            lax.fori_loop(0, GH, _f, 0)

        def _drain_g(p, half, rbuf):
            def _f(j, cc):
                pltpu.make_async_copy(g_hbm.at[ibs.at[p, half * GH + j]],
                                      rbuf.at[j], gsem).wait()
                return cc
            lax.fori_loop(0, GH, _f, 0)

        def _fire_s(p, half, rbuf):
            def _f(j, cc):
                pltpu.async_copy(rbuf.at[j], acc.at[ibd.at[p, half * GH + j]],
                                 ssem, add=True)
                return cc
            lax.fori_loop(0, GH, _f, 0)

        def _drain_s(p, half, rbuf):
            def _f(j, cc):
                pltpu.make_async_copy(rbuf.at[j], acc.at[ibd.at[p, half * GH + j]],
                                      ssem).wait()
                return cc
            lax.fori_loop(0, GH, _f, 0)

        _load_idx(0, 0)
        _fire_g(0, 0, rows_a)

        def _pair(i, carry):
            for p, off in ((0, 0), (1, 1)):
                b = 2 * i + off
                _drain_g(p, 0, rows_a)

                @pl.when(b > 0)
                def _():
                    _drain_s(1 - p, 1, rows_b)

                _fire_s(p, 0, rows_a)
                _fire_g(p, 1, rows_b)

                @pl.when(b + 1 < NB)
                def _():
                    _load_idx(b + 1, 1 - p)

                _drain_g(p, 1, rows_b)
                _drain_s(p, 0, rows_a)

                _fire_s(p, 1, rows_b)

                @pl.when(b + 1 < NB)
                def _():
                    _fire_g(1 - p, 0, rows_a)
            return carry

        lax.fori_loop(0, NB // 2, _pair, 0)
        _drain_s(1, 1, rows_b)
        plsc.subcore_barrier()

        for k in range(NKC):
            sl = pl.ds(nbase + k * NCH, NCH)
            pltpu.sync_copy(acc.at[sl], obuf)
            pltpu.sync_copy(obuf, p_hbm.at[sl])

    @pl.when(c == 0)
    def _():
        _run(ga_hbm, pa_hbm)

    @pl.when(c == 1)
    def _():
        _run(gb_hbm, pb_hbm)


R = 2000


def _k1_body(x_ref, dga_ref, dgb_ref, w1_ref, ga_ref, gb_ref):
    d = lax.rsqrt(dga_ref[:, 0] + dgb_ref[:, 0] + 1.0)
    h = jnp.dot(x_ref[...], w1_ref[...], preferred_element_type=jnp.float32)
    g = h * d[:, None]
    ga_ref[...] = g[:, :HALF]
    gb_ref[...] = g[:, HALF:]


def _mid_body(pa_ref, pb_ref, ga_ref, gb_ref, dga_ref, dgb_ref, b_ref, w_ref,
              oa_ref, ob_ref):
    d = lax.rsqrt(dga_ref[:, 0] + dgb_ref[:, 0] + 1.0)
    za = jnp.maximum((pa_ref[...] + ga_ref[...]) * d[:, None] + b_ref[0, :HALF], 0.0)
    zb = jnp.maximum((pb_ref[...] + gb_ref[...]) * d[:, None] + b_ref[0, HALF:], 0.0)
    z = jnp.concatenate([za, zb], axis=1)
    h = jnp.dot(z, w_ref[...], preferred_element_type=jnp.float32)
    g = h * d[:, None]
    oa_ref[...] = g[:, :HALF]
    ob_ref[...] = g[:, HALF:]


def _head_body(pa_ref, pb_ref, ga_ref, gb_ref, dga_ref, dgb_ref, b2_ref,
               wo1_ref, bo1_ref, wo2_ref, bo2_ref, y_ref):
    d = lax.rsqrt(dga_ref[:, 0] + dgb_ref[:, 0] + 1.0)
    za = jnp.maximum((pa_ref[...] + ga_ref[...]) * d[:, None] + b2_ref[0, :HALF], 0.0)
    zb = jnp.maximum((pb_ref[...] + gb_ref[...]) * d[:, None] + b2_ref[0, HALF:], 0.0)
    z = jnp.concatenate([za, zb], axis=1)
    m = jnp.dot(z, wo1_ref[...], preferred_element_type=jnp.float32) + bo1_ref[0, :]
    m = jnp.where(m > 0.0, m, jnp.exp(jnp.minimum(m, 0.0)) - 1.0)
    y_ref[...] = jnp.dot(m, wo2_ref[...], preferred_element_type=jnp.float32) + bo2_ref[0, :]


def _row_spec(cols):
    return pl.BlockSpec((R, cols), lambda i: (i, 0))


def _full_spec(shape):
    nd = len(shape)
    return pl.BlockSpec(shape, lambda i: (0,) * nd)


def kernel(x, edge_index, batch, W1, b1, W2, b2, Wo1, bo1, Wo2, bo2):
    src = edge_index[0]
    dst = edge_index[1]
    src2d = jnp.concatenate([src, jnp.zeros((EPAD,), jnp.int32)]).reshape(ROWS, LANES)
    dst2d = jnp.concatenate([dst, jnp.full((EPAD,), N, jnp.int32)]).reshape(ROWS, LANES)

    dga, dgb = _deg_kernel(dst2d)
    dga = dga.reshape(N, 1)
    dgb = dgb.reshape(N, 1)

    ga, gb = pl.pallas_call(
        _k1_body,
        grid=(N // R,),
        in_specs=[_row_spec(IN_DIM), _row_spec(1), _row_spec(1),
                  _full_spec((IN_DIM, HID))],
        out_specs=[_row_spec(HALF), _row_spec(HALF)],
        out_shape=[jax.ShapeDtypeStruct((N, HALF), jnp.float32),
                   jax.ShapeDtypeStruct((N, HALF), jnp.float32)],
    )(x, dga, dgb, W1)

    pa, pb = _prop_kernel(src2d, dst2d, ga, gb)

    g2a, g2b = pl.pallas_call(
        _mid_body,
        grid=(N // R,),
        in_specs=[_row_spec(HALF), _row_spec(HALF), _row_spec(HALF),
                  _row_spec(HALF), _row_spec(1), _row_spec(1),
                  _full_spec((1, HID)), _full_spec((HID, HID))],
        out_specs=[_row_spec(HALF), _row_spec(HALF)],
        out_shape=[jax.ShapeDtypeStruct((N, HALF), jnp.float32),
                   jax.ShapeDtypeStruct((N, HALF), jnp.float32)],
    )(pa, pb, ga, gb, dga, dgb, b1.reshape(1, HID), W2)

    pa2, pb2 = _prop_kernel(src2d, dst2d, g2a, g2b)

    y = pl.pallas_call(
        _head_body,
        grid=(N // R,),
        in_specs=[_row_spec(HALF), _row_spec(HALF), _row_spec(HALF),
                  _row_spec(HALF), _row_spec(1), _row_spec(1),
                  _full_spec((1, HID)), _full_spec((HID, HALF)),
                  _full_spec((1, HALF)), _full_spec((HALF, OUT_DIM)),
                  _full_spec((1, OUT_DIM))],
        out_specs=[_row_spec(OUT_DIM)],
        out_shape=[jax.ShapeDtypeStruct((N, OUT_DIM), jnp.float32)],
    )(pa2, pb2, g2a, g2b, dga, dgb, b2.reshape(1, HID),
      Wo1, bo1.reshape(1, HALF), Wo2, bo2.reshape(1, OUT_DIM))[0]

    return jnp.squeeze(y, -1)


# speedup vs baseline: 30.9943x; 1.2846x over previous
import functools

import jax
import jax.numpy as jnp
from jax import lax
from jax.experimental import pallas as pl
from jax.experimental.pallas import tpu as pltpu
from jax.experimental.pallas import tpu_sc as plsc

N = 100000
E = 1600000
IN_DIM = 11
HID = 32
HALF = 16
OUT_DIM = 1
WIDE = 128

NC = 2
NS = 16
LANES = 128
ROWS = 12800
EPAD = ROWS * LANES - E

RPT = ROWS // NS
RPT_D = ROWS // (NC * NS)

GB = 8
GH = 4
NB = RPT // GB
NB_D = RPT_D // GB

NPAD = 16
NPT = 6256
NCH = 368
NKC = NPT // NCH

_mesh = plsc.VectorSubcoreMesh(
    core_axis_name="c", subcore_axis_name="s", num_cores=NC, num_subcores=NS
)


@functools.partial(
    pl.kernel,
    out_type=[jax.ShapeDtypeStruct((N,), jnp.float32),
              jax.ShapeDtypeStruct((N,), jnp.float32)],
    mesh=_mesh,
    scratch_types=[
        pltpu.VMEM((2, GB, LANES), jnp.int32),
        pltpu.VMEM((LANES,), jnp.float32),
        pltpu.VMEM((NCH,), jnp.float32),
        pltpu.VMEM_SHARED((N + NPAD,), jnp.float32),
        pltpu.SemaphoreType.DMA,
    ],
    compiler_params=pltpu.CompilerParams(use_tc_tiling_on_sc=False),
)
def _deg_kernel(dst_hbm, outa_hbm, outb_hbm, ibd, ones, obuf, dacc, ssem):
    c = lax.axis_index("c")
    s = lax.axis_index("s")
    nbase = jnp.minimum(s * NPT, N - NPT)

    def _zero(i, carry):
        obuf[pl.ds(i * 16, 16)] = jnp.zeros((16,), jnp.float32)
        return carry

    lax.fori_loop(0, NCH // 16, _zero, 0)

    def _one(i, carry):
        ones[pl.ds(i * 16, 16)] = jnp.ones((16,), jnp.float32)
        return carry

    lax.fori_loop(0, LANES // 16, _one, 0)

    for k in range(NKC):
        pltpu.sync_copy(obuf, dacc.at[pl.ds(nbase + k * NCH, NCH)])
    plsc.subcore_barrier()

    rbase = (c * NS + s) * RPT_D

    def _fire(p, half):
        def _f(j, cc):
            pltpu.async_copy(ones, dacc.at[ibd.at[p, half * GH + j]],
                             ssem, add=True)
            return cc
        lax.fori_loop(0, GH, _f, 0)

    def _drain(p, half):
        def _f(j, cc):
            pltpu.make_async_copy(ones, dacc.at[ibd.at[p, half * GH + j]],
                                  ssem).wait()
            return cc
        lax.fori_loop(0, GH, _f, 0)

    def _load_idx(b, p):
        pltpu.sync_copy(dst_hbm.at[pl.ds(rbase + b * GB, GB)], ibd.at[p])

    _load_idx(0, 0)

    def _pair(i, carry):
        for p, off in ((0, 0), (1, 1)):
            b = 2 * i + off
            _fire(p, 0)

            @pl.when(b > 0)
            def _():
                _drain(1 - p, 1)

            _fire(p, 1)

            @pl.when(b + 1 < NB_D)
            def _():
                _load_idx(b + 1, 1 - p)

            _drain(p, 0)
        return carry

    lax.fori_loop(0, NB_D // 2, _pair, 0)
    _drain(1, 1)
    plsc.subcore_barrier()

    for k in range(NKC):
        sl = pl.ds(nbase + k * NCH, NCH)
        pltpu.sync_copy(dacc.at[sl], obuf)

        @pl.when(c == 0)
        def _():
            pltpu.sync_copy(obuf, outa_hbm.at[sl])

        @pl.when(c == 1)
        def _():
            pltpu.sync_copy(obuf, outb_hbm.at[sl])


@functools.partial(
    pl.kernel,
    out_type=[jax.ShapeDtypeStruct((N, WIDE), jnp.float32),
              jax.ShapeDtypeStruct((N, HALF), jnp.float32),
              jax.ShapeDtypeStruct((N, HALF), jnp.float32)],
    mesh=_mesh,
    scratch_types=[
        pltpu.VMEM((2, GB, LANES), jnp.int32),
        pltpu.VMEM((2, GB, LANES), jnp.int32),
        pltpu.VMEM((GH, LANES, HALF), jnp.float32),
        pltpu.VMEM((GH, LANES, HALF), jnp.float32),
        pltpu.VMEM((NCH, HALF), jnp.float32),
        pltpu.VMEM((NCH,), jnp.float32),
        pltpu.VMEM((16,), jnp.float32),
        pltpu.VMEM_SHARED((N + NPAD, HALF), jnp.float32),
        pltpu.SemaphoreType.DMA,
        pltpu.SemaphoreType.DMA,
    ],
    compiler_params=pltpu.CompilerParams(use_tc_tiling_on_sc=False),
)
def _prop_kernel(src_hbm, dst_hbm, h_hbm, d_hbm, b_hbm,
                 z_hbm, gta_hbm, gtb_hbm,
                 ibs, ibd, rows_a, rows_b, hbuf, dbuf, bbuf, acc, gsem, ssem):
    c = lax.axis_index("c")
    s = lax.axis_index("s")
    nbase = jnp.minimum(s * NPT, N - NPT)

    def _run(col, gt_hbm):
        pltpu.sync_copy(b_hbm.at[pl.ds(col, HALF)], bbuf)

        def _p0(k, carry):
            sl = pl.ds(nbase + k * NCH, NCH)
            pltpu.sync_copy(h_hbm.at[sl, pl.ds(col, HALF)], hbuf)
            pltpu.sync_copy(d_hbm.at[sl], dbuf)

            def _grp(r, cc):
                dv16 = dbuf[pl.ds(r * 16, 16)]
                for j in range(16):
                    row = r * 16 + j
                    hbuf[row, :] = hbuf[row, :] * jnp.full((16,), dv16[j],
                                                           jnp.float32)
                return cc

            lax.fori_loop(0, NCH // 16, _grp, 0)
            pltpu.sync_copy(hbuf, gt_hbm.at[sl])
            pltpu.sync_copy(hbuf, acc.at[sl])
            return carry

        lax.fori_loop(0, NKC, _p0, 0)
        plsc.subcore_barrier()

        rbase = s * RPT

        def _load_idx(b, p):
            pltpu.sync_copy(src_hbm.at[pl.ds(rbase + b * GB, GB)], ibs.at[p])
            pltpu.sync_copy(dst_hbm.at[pl.ds(rbase + b * GB, GB)], ibd.at[p])

        def _fire_g(p, half, rbuf):
            def _f(j, cc):
                pltpu.async_copy(gt_hbm.at[ibs.at[p, half * GH + j]],
                                 rbuf.at[j], gsem)
                return cc
            lax.fori_loop(0, GH, _f, 0)

        def _drain_g(p, half, rbuf):
            def _f(j, cc):
                pltpu.make_async_copy(gt_hbm.at[ibs.at[p, half * GH + j]],
                                      rbuf.at[j], gsem).wait()
                return cc
            lax.fori_loop(0, GH, _f, 0)

        def _fire_s(p, half, rbuf):
            def _f(j, cc):
                pltpu.async_copy(rbuf.at[j], acc.at[ibd.at[p, half * GH + j]],
                                 ssem, add=True)
                return cc
            lax.fori_loop(0, GH, _f, 0)

        def _drain_s(p, half, rbuf):
            def _f(j, cc):
                pltpu.make_async_copy(rbuf.at[j],
                                      acc.at[ibd.at[p, half * GH + j]],
                                      ssem).wait()
                return cc
            lax.fori_loop(0, GH, _f, 0)

        _load_idx(0, 0)
        _fire_g(0, 0, rows_a)

        def _pair(i, carry):
            for p, off in ((0, 0), (1, 1)):
                b = 2 * i + off
                _drain_g(p, 0, rows_a)

                @pl.when(b > 0)
                def _():
                    _drain_s(1 - p, 1, rows_b)

                _fire_s(p, 0, rows_a)
                _fire_g(p, 1, rows_b)

                @pl.when(b + 1 < NB)
                def _():
                    _load_idx(b + 1, 1 - p)

                _drain_g(p, 1, rows_b)
                _drain_s(p, 0, rows_a)

                _fire_s(p, 1, rows_b)

                @pl.when(b + 1 < NB)
                def _():
                    _fire_g(1 - p, 0, rows_a)
            return carry

        lax.fori_loop(0, NB // 2, _pair, 0)
        _drain_s(1, 1, rows_b)
        plsc.subcore_barrier()

        bv = bbuf[...]

        def _p2(k, carry):
            sl = pl.ds(nbase + k * NCH, NCH)
            pltpu.sync_copy(acc.at[sl], hbuf)
            pltpu.sync_copy(d_hbm.at[sl], dbuf)

            def _grp(r, cc):
                dv16 = dbuf[pl.ds(r * 16, 16)]
                for j in range(16):
                    row = r * 16 + j
                    v = hbuf[row, :] * jnp.full((16,), dv16[j], jnp.float32) + bv
                    hbuf[row, :] = jnp.maximum(v, 0.0)
                return cc

            lax.fori_loop(0, NCH // 16, _grp, 0)
            pltpu.sync_copy(hbuf, z_hbm.at[sl, pl.ds(col, HALF)])
            return carry

        lax.fori_loop(0, NKC, _p2, 0)

    @pl.when(c == 0)
    def _():
        _run(0, gta_hbm)

    @pl.when(c == 1)
    def _():
        _run(HALF, gtb_hbm)


R = 2048
NG = (N + R - 1) // R


def _h1_body(x_ref, w1_ref, h_ref):
    h_ref[:, :HID] = jnp.dot(x_ref[...], w1_ref[...],
                             preferred_element_type=jnp.float32)


def _d_body(dga_ref, dgb_ref, d_ref):
    d_ref[...] = lax.rsqrt(dga_ref[...] + dgb_ref[...] + 1.0)


def _mid_body(z_ref, w_ref, o_ref):
    o_ref[:, :HID] = jnp.dot(z_ref[:, :HID], w_ref[...],
                             preferred_element_type=jnp.float32)


def _head_body(z_ref, wo1_ref, bo1_ref, wo2_ref, bo2_ref, y_ref):
    m = jnp.dot(z_ref[:, :HID], wo1_ref[...],
                preferred_element_type=jnp.float32) + bo1_ref[...][None, :]
    m = jnp.where(m > 0.0, m, jnp.exp(jnp.minimum(m, 0.0)) - 1.0)
    y_ref[...] = jnp.dot(m, wo2_ref[...],
                         preferred_element_type=jnp.float32)[:, 0] + bo2_ref[0]


def _row_spec(cols):
    return pl.BlockSpec((R, cols), lambda i: (i, 0))


def _vec_spec():
    return pl.BlockSpec((R,), lambda i: (i,))


def _full_spec(shape):
    nd = len(shape)
    return pl.BlockSpec(shape, lambda i: (0,) * nd)


def kernel(x, edge_index, batch, W1, b1, W2, b2, Wo1, bo1, Wo2, bo2):
    src = edge_index[0]
    dst = edge_index[1]
    src2d = jnp.concatenate([src, jnp.zeros((EPAD,), jnp.int32)]).reshape(ROWS, LANES)
    dst2d = jnp.concatenate([dst, jnp.full((EPAD,), N, jnp.int32)]).reshape(ROWS, LANES)

    dga, dgb = _deg_kernel(dst2d)

    h1 = pl.pallas_call(
        _h1_body,
        grid=(NG,),
        in_specs=[_row_spec(IN_DIM), _full_spec((IN_DIM, HID))],
        out_specs=[_row_spec(WIDE)],
        out_shape=[jax.ShapeDtypeStruct((N, WIDE), jnp.float32)],
    )(x, W1)[0]

    d = pl.pallas_call(
        _d_body,
        grid=(NG,),
        in_specs=[_vec_spec(), _vec_spec()],
        out_specs=[_vec_spec()],
        out_shape=[jax.ShapeDtypeStruct((N,), jnp.float32)],
    )(dga, dgb)[0]

    z1 = _prop_kernel(src2d, dst2d, h1, d, b1)[0]

    h2 = pl.pallas_call(
        _mid_body,
        grid=(NG,),
        in_specs=[_row_spec(WIDE), _full_spec((HID, HID))],
        out_specs=[_row_spec(WIDE)],
        out_shape=[jax.ShapeDtypeStruct((N, WIDE), jnp.float32)],
    )(z1, W2)[0]

    z2 = _prop_kernel(src2d, dst2d, h2, d, b2)[0]

    y = pl.pallas_call(
        _head_body,
        grid=(NG,),
        in_specs=[_row_spec(WIDE), _full_spec((HID, HALF)), _full_spec((HALF,)),
                  _full_spec((HALF, OUT_DIM)), _full_spec((OUT_DIM,))],
        out_specs=[_vec_spec()],
        out_shape=[jax.ShapeDtypeStruct((N,), jnp.float32)],
    )(z2, Wo1, bo1, Wo2, bo2)[0]

    return y
